# Initial kernel scaffold; baseline (speedup 1.0000x reference)
import jax
import jax.numpy as jnp
from jax.experimental import pallas as pl


def kernel(emb_V, emb_E, edge_index, W_v2e, b_v2e, W_e2v, b_e2v, W_fuse, b_fuse):
    raise NotImplementedError("write your pallas kernel here")


# speedup vs baseline: 5.1779x; 5.1779x over previous
import jax
import jax.numpy as jnp
from jax import lax
from jax.experimental import pallas as pl
from jax.experimental.pallas import tpu as pltpu
from jax.experimental.pallas import tpu_sc as plsc

D = 128
L = 16
_NC, _NS = 2, 16
_NW = _NC * _NS

_ROW_BLOCK = 1000


def _linear_body(x_ref, w_ref, b_ref, o_ref):
    y = jnp.dot(x_ref[...], w_ref[...], preferred_element_type=jnp.float32)
    o_ref[...] = y + b_ref[...]


def _tc_linear(x, w, b):
    n, d = x.shape
    return pl.pallas_call(
        _linear_body,
        grid=(n // _ROW_BLOCK,),
        in_specs=[
            pl.BlockSpec((_ROW_BLOCK, d), lambda i: (i, 0)),
            pl.BlockSpec((d, D), lambda i: (0, 0)),
            pl.BlockSpec((1, D), lambda i: (0, 0)),
        ],
        out_specs=pl.BlockSpec((_ROW_BLOCK, D), lambda i: (i, 0)),
        out_shape=jax.ShapeDtypeStruct((n, D), jnp.float32),
    )(x, w, b.reshape(1, D))


def _fuse_body(s_ref, c_ref, embE_ref, wf_ref, bf_ref, we_ref, be_ref,
               eo_ref, y_ref):
    s = s_ref[0] + s_ref[1]
    cnt = jnp.maximum(c_ref[0] + c_ref[1], 1.0)
    tem = jnp.maximum(s / cnt, 0.0)
    eo = jnp.dot(embE_ref[...], wf_ref[:D, :], preferred_element_type=jnp.float32)
    eo = eo + jnp.dot(tem, wf_ref[D:, :], preferred_element_type=jnp.float32)
    eo = eo + bf_ref[...]
    eo_ref[...] = eo
    y_ref[...] = jnp.dot(eo, we_ref[...], preferred_element_type=jnp.float32) + be_ref[...]


def _tc_fuse(s, cnt, emb_E, w_fuse, b_fuse, w_e2v, b_e2v):
    n = emb_E.shape[0]
    return pl.pallas_call(
        _fuse_body,
        grid=(n // _ROW_BLOCK,),
        in_specs=[
            pl.BlockSpec((_NC, _ROW_BLOCK, D), lambda i: (0, i, 0)),
            pl.BlockSpec((_NC, _ROW_BLOCK, 1), lambda i: (0, i, 0)),
            pl.BlockSpec((_ROW_BLOCK, D), lambda i: (i, 0)),
            pl.BlockSpec((2 * D, D), lambda i: (0, 0)),
            pl.BlockSpec((1, D), lambda i: (0, 0)),
            pl.BlockSpec((D, D), lambda i: (0, 0)),
            pl.BlockSpec((1, D), lambda i: (0, 0)),
        ],
        out_specs=[
            pl.BlockSpec((_ROW_BLOCK, D), lambda i: (i, 0)),
            pl.BlockSpec((_ROW_BLOCK, D), lambda i: (i, 0)),
        ],
        out_shape=[
            jax.ShapeDtypeStruct((n, D), jnp.float32),
            jax.ShapeDtypeStruct((n, D), jnp.float32),
        ],
    )(s, cnt, emb_E, w_fuse, b_fuse.reshape(1, D), w_e2v, b_e2v.reshape(1, D))


def _finish_body(t_ref, c_ref, o_ref):
    t = t_ref[0] + t_ref[1]
    cnt = jnp.maximum(c_ref[0] + c_ref[1], 1.0)
    o_ref[...] = jnp.maximum(t / cnt, 0.0)


def _tc_finish(t, cnt, n):
    return pl.pallas_call(
        _finish_body,
        grid=(n // _ROW_BLOCK,),
        in_specs=[
            pl.BlockSpec((_NC, _ROW_BLOCK, D), lambda i: (0, i, 0)),
            pl.BlockSpec((_NC, _ROW_BLOCK, 1), lambda i: (0, i, 0)),
        ],
        out_specs=pl.BlockSpec((_ROW_BLOCK, D), lambda i: (i, 0)),
        out_shape=jax.ShapeDtypeStruct((n, D), jnp.float32),
    )(t, cnt)


def _sc_segment_sum(x, src_idx, dst_idx, num_seg, zeros_pad, chunk=80):
    e = src_idx.shape[0]
    epw = e // _NW
    nchunk = epw // chunk
    seg_pad = ((num_seg + 8 * _NS - 1) // (8 * _NS)) * (8 * _NS)
    rpw = seg_pad // _NS
    crows = (num_seg + D - 1) // D
    crows = ((crows + 7) // 8) * 8
    mesh = plsc.VectorSubcoreMesh(core_axis_name="c", subcore_axis_name="s")

    @pl.kernel(
        out_type=[
            jax.ShapeDtypeStruct((_NC, seg_pad, D), jnp.float32),
            jax.ShapeDtypeStruct((_NC, crows, D), jnp.float32),
        ],
        mesh=mesh,
        scratch_types=[
            pltpu.VMEM((chunk,), jnp.int32),
            pltpu.VMEM((chunk,), jnp.int32),
            pltpu.VMEM((chunk, D), jnp.float32),
            pltpu.VMEM((crows, D), jnp.float32),
            pltpu.VMEM((crows,), jnp.int32),
            pltpu.VMEM_SHARED((seg_pad, D), jnp.float32),
            pltpu.VMEM_SHARED((crows, D), jnp.float32),
            pltpu.SemaphoreType.DMA,
        ],
        compiler_params=pltpu.CompilerParams(needs_layout_passes=False),
    )
    def k(x_hbm, src_hbm, dst_hbm, z_hbm, sum_hbm, cnt_hbm,
          src_v, dst_v, rows_v, hist_v, ident_v, acc_sh, cnt_sh, sem):
        cid = lax.axis_index("c")
        sid = lax.axis_index("s")
        wid = sid * _NC + cid
        pltpu.sync_copy(z_hbm.at[pl.ds(sid * rpw, rpw)],
                        acc_sh.at[pl.ds(sid * rpw, rpw)])
        pltpu.sync_copy(z_hbm.at[pl.ds(0, crows)], hist_v)

        @pl.when(sid == 0)
        def _():
            pltpu.sync_copy(z_hbm.at[pl.ds(0, crows)], cnt_sh)

        @pl.loop(0, crows // L)
        def _(j):
            ident_v[pl.ds(j * L, L)] = lax.iota(jnp.int32, L) + j * L

        plsc.subcore_barrier()

        base = wid * epw
        ones = jnp.ones((L,), jnp.float32)

        @pl.loop(0, nchunk)
        def _(i):
            off = base + i * chunk
            pltpu.sync_copy(src_hbm.at[pl.ds(off, chunk)], src_v)
            pltpu.sync_copy(dst_hbm.at[pl.ds(off, chunk)], dst_v)
            pltpu.async_copy(x_hbm.at[src_v], rows_v, sem).wait()
            pltpu.sync_copy(rows_v, acc_sh.at[dst_v], add=True)

            @pl.loop(0, chunk // L)
            def _(j):
                d16 = dst_v[pl.ds(j * L, L)]
                row = lax.shift_right_logical(d16, 7)
                lane = lax.bitwise_and(d16, 127)
                plsc.addupdate_scatter(hist_v, [row, lane], ones)

        pltpu.sync_copy(hist_v, cnt_sh.at[ident_v], add=True)
        plsc.subcore_barrier()
        pltpu.sync_copy(acc_sh.at[pl.ds(sid * rpw, rpw)],
                        sum_hbm.at[cid, pl.ds(sid * rpw, rpw)])

        @pl.when(sid == 0)
        def _():
            pltpu.sync_copy(cnt_sh, cnt_hbm.at[cid])

    return k(x, src_idx, dst_idx, zeros_pad)


def kernel(emb_V, emb_E, edge_index, W_v2e, b_v2e, W_e2v, b_e2v, W_fuse,
           b_fuse):
    n_v = emb_V.shape[0]
    n_e = emb_E.shape[0]
    src = edge_index[0].astype(jnp.int32)
    dst = edge_index[1].astype(jnp.int32)

    seg_pad_e = ((n_e + 8 * _NS - 1) // (8 * _NS)) * (8 * _NS)
    seg_pad_v = ((n_v + 8 * _NS - 1) // (8 * _NS)) * (8 * _NS)
    zeros_e = jnp.zeros((seg_pad_e, D), jnp.float32)
    zeros_v = jnp.zeros((seg_pad_v, D), jnp.float32)

    x = _tc_linear(emb_V, W_v2e, b_v2e)
    s, cnt_e = _sc_segment_sum(x, src, dst, n_e, zeros_e)
    cnt_e = cnt_e.reshape(_NC, -1, 1)
    emb_E_out, y = _tc_fuse(s, cnt_e, emb_E, W_fuse, b_fuse, W_e2v, b_e2v)
    t, cnt_v = _sc_segment_sum(y, dst, src, n_v, zeros_v)
    cnt_v = cnt_v.reshape(_NC, -1, 1)
    emb_V_out = _tc_finish(t, cnt_v, n_v)
    return (emb_V_out, emb_E_out)
